# Initial kernel scaffold; baseline (speedup 1.0000x reference)
import jax
import jax.numpy as jnp
from jax.experimental import pallas as pl


def kernel(x, emb_proto, emb_flags, emb_dir, W_len, b_len, W_iat, b_iat, W_fus, b_fus, gamma, beta):
    raise NotImplementedError("write your pallas kernel here")


# speedup vs baseline: 3.4052x; 3.4052x over previous
import functools

import jax
import jax.numpy as jnp
from jax.experimental import pallas as pl
from jax.experimental.pallas import tpu as pltpu

B, L = 4096, 50
N = B * L
DE, DM = 32, 256
BT = 512


def _fold_kernel(emb_proto_ref, emb_flags_ref, emb_dir_ref, W_len_ref, b_len_ref,
                 W_iat_ref, b_iat_ref, W_fus_ref, b_fus_ref, gamma_ref, beta_ref,
                 Tp_ref, Tf_ref, smalls_ref):
    Wf = W_fus_ref[:, :]
    Wp = Wf[:, 0:DE]
    Wl = Wf[:, DE:2 * DE]
    Wfl = Wf[:, 2 * DE:3 * DE]
    Wi = Wf[:, 3 * DE:4 * DE]
    Wd = Wf[:, 4 * DE:4 * DE + DE // 4]
    Tp_ref[:, :] = jax.lax.dot_general(
        emb_proto_ref[:, :], Wp, (((1,), (1,)), ((), ())),
        preferred_element_type=jnp.float32)
    Tf_ref[:, :] = jax.lax.dot_general(
        emb_flags_ref[:, :], Wfl, (((1,), (1,)), ((), ())),
        preferred_element_type=jnp.float32)
    v_len = jnp.dot(Wl, W_len_ref[:, 0], preferred_element_type=jnp.float32)
    v_iat = jnp.dot(Wi, W_iat_ref[:, 0], preferred_element_type=jnp.float32)
    c0 = (b_fus_ref[:] + jnp.dot(Wl, b_len_ref[:], preferred_element_type=jnp.float32)
          + jnp.dot(Wi, b_iat_ref[:], preferred_element_type=jnp.float32))
    ed = emb_dir_ref[:, :]
    Td = jax.lax.dot_general(ed, Wd, (((1,), (1,)), ((), ())),
                             preferred_element_type=jnp.float32)
    smalls_ref[0, :] = v_len
    smalls_ref[1, :] = v_iat
    smalls_ref[2, :] = Td[0, :] + c0
    smalls_ref[3, :] = Td[1, :] - Td[0, :]
    smalls_ref[4, :] = gamma_ref[:]
    smalls_ref[5, :] = beta_ref[:]
    smalls_ref[6, :] = jnp.zeros((DM,), jnp.float32)
    smalls_ref[7, :] = jnp.zeros((DM,), jnp.float32)


def _fold(emb_proto, emb_flags, emb_dir, W_len, b_len, W_iat, b_iat, W_fus,
          b_fus, gamma, beta):
    return pl.pallas_call(
        _fold_kernel,
        out_shape=(
            jax.ShapeDtypeStruct((256, DM), jnp.float32),
            jax.ShapeDtypeStruct((64, DM), jnp.float32),
            jax.ShapeDtypeStruct((8, DM), jnp.float32),
        ),
    )(emb_proto, emb_flags, emb_dir, W_len, b_len, W_iat, b_iat, W_fus,
      b_fus, gamma, beta)


def _main_kernel(x_ref, Tp_ref, Tf_ref, smalls_ref, out_ref):
    xb = x_ref[:, :]
    xi = xb.astype(jnp.int32)
    p = jnp.clip(xi[:, 0], 0, 255)
    f = jnp.clip(xi[:, 2], 0, 63)
    d = jnp.clip(xi[:, 4], 0, 1).astype(jnp.float32)
    x1 = xb[:, 1]
    x3 = xb[:, 3]
    iota_p = jax.lax.broadcasted_iota(jnp.int32, (BT, 256), 1)
    oh_p = (p[:, None] == iota_p).astype(jnp.float32)
    iota_f = jax.lax.broadcasted_iota(jnp.int32, (BT, 64), 1)
    oh_f = (f[:, None] == iota_f).astype(jnp.float32)
    h = jax.lax.dot_general(oh_p, Tp_ref[:, :], (((1,), (0,)), ((), ())),
                            preferred_element_type=jnp.float32)
    h += jax.lax.dot_general(oh_f, Tf_ref[:, :], (((1,), (0,)), ((), ())),
                             preferred_element_type=jnp.float32)
    h += smalls_ref[2, :][None, :]
    h += d[:, None] * smalls_ref[3, :][None, :]
    h += x1[:, None] * smalls_ref[0, :][None, :]
    h += x3[:, None] * smalls_ref[1, :][None, :]
    mu = jnp.mean(h, axis=1, keepdims=True)
    hc = h - mu
    var = jnp.mean(hc * hc, axis=1, keepdims=True)
    s = jax.lax.rsqrt(var + 1e-5)
    out_ref[:, :] = hc * s * smalls_ref[4, :][None, :] + smalls_ref[5, :][None, :]


@jax.jit
def kernel(x, emb_proto, emb_flags, emb_dir, W_len, b_len, W_iat, b_iat,
           W_fus, b_fus, gamma, beta):
    Tp, Tf, smalls = _fold(emb_proto, emb_flags, emb_dir, W_len, b_len,
                           W_iat, b_iat, W_fus, b_fus, gamma, beta)
    xr = x.reshape(N, 5)
    out = pl.pallas_call(
        _main_kernel,
        grid=(N // BT,),
        in_specs=[
            pl.BlockSpec((BT, 5), lambda i: (i, 0)),
            pl.BlockSpec((256, DM), lambda i: (0, 0)),
            pl.BlockSpec((64, DM), lambda i: (0, 0)),
            pl.BlockSpec((8, DM), lambda i: (0, 0)),
        ],
        out_specs=pl.BlockSpec((BT, DM), lambda i: (i, 0)),
        out_shape=jax.ShapeDtypeStruct((N, DM), jnp.float32),
    )(xr, Tp, Tf, smalls)
    return out.reshape(B, L, DM)
